# baseline (speedup 1.0000x reference)
import jax
import jax.numpy as jnp
from jax import lax
from jax.experimental import pallas as pl
from jax.experimental.pallas import tpu as pltpu
from jax.experimental.pallas import tpu_sc as plsc

FEATS_TOTAL = 16384
N_ROWS = 768
HID = 256
NT = 16
IDX_PER_TILE = FEATS_TOTAL // NT
ROWS_PER_TILE = N_ROWS // NT
LANES = 16
HB = 4


def _dyn_take(v, idx):
    dnums = lax.GatherDimensionNumbers(
        offset_dims=(), collapsed_slice_dims=(0,), start_index_map=(0,))
    return lax.gather(v, idx[:, None], dnums, slice_sizes=(1,),
                      mode=lax.GatherScatterMode.PROMISE_IN_BOUNDS)


def _nnue_body(feats, w1f, b1, w2w, w2bp, out_hbm,
               idx_v, hist_v, hs_v, cw_v, w1_v, xpart_v, xs_v,
               b1s_v, w2ws_v, pv_v, ps_v, w2b_v, out_v,
               hist_sh, xpart_sh, prod_sh, w1_sem, ex_sem):
    sid = lax.axis_index("s")
    lane_iota = lax.iota(jnp.int32, LANES)

    w1_cp = pltpu.async_copy(
        w1f.at[pl.ds(sid * ROWS_PER_TILE * HID, ROWS_PER_TILE * HID)],
        w1_v, w1_sem)

    zeros16 = jnp.zeros((LANES,), jnp.float32)
    for i in range(N_ROWS // LANES):
        hist_v[pl.ds(i * LANES, LANES)] = zeros16

    pltpu.sync_copy(feats.at[pl.ds(sid * IDX_PER_TILE, IDX_PER_TILE)],
                    idx_v)
    ones16 = jnp.full((LANES,), 1.0, jnp.float32)
    for j in range(IDX_PER_TILE // LANES):
        plsc.addupdate_scatter(hist_v, [idx_v[pl.ds(j * LANES, LANES)]],
                               ones16)
    pltpu.sync_copy(hist_v, hist_sh.at[pl.ds(sid * N_ROWS, N_ROWS)])

    plsc.subcore_barrier()

    cps = [pltpu.async_copy(
        hist_sh.at[pl.ds(i * N_ROWS + sid * ROWS_PER_TILE, ROWS_PER_TILE)],
        hs_v.at[pl.ds(i * ROWS_PER_TILE, ROWS_PER_TILE)], ex_sem)
        for i in range(NT)]
    for cp in cps:
        cp.wait()
    for c in range(ROWS_PER_TILE // LANES):
        acc = hs_v[pl.ds(c * LANES, LANES)]
        for i in range(1, NT):
            acc = acc + hs_v[pl.ds(i * ROWS_PER_TILE + c * LANES, LANES)]
        cw_v[pl.ds(c * LANES, LANES)] = acc

    w1_cp.wait()
    chunks = [cw_v[pl.ds(c * LANES, LANES)]
              for c in range(ROWS_PER_TILE // LANES)]
    for hb in range(HID // LANES // HB):
        accs = [jnp.zeros((LANES,), jnp.float32) for _ in range(HB)]
        for c in range(ROWS_PER_TILE // LANES):
            for r in range(LANES):
                bc = _dyn_take(chunks[c], jnp.full((LANES,), r, jnp.int32))
                base = (c * LANES + r) * HID
                for hh in range(HB):
                    accs[hh] = accs[hh] + bc * w1_v[
                        pl.ds(base + (hb * HB + hh) * LANES, LANES)]
        for hh in range(HB):
            xpart_v[pl.ds((hb * HB + hh) * LANES, LANES)] = accs[hh]
    pltpu.sync_copy(xpart_v, xpart_sh.at[pl.ds(sid * HID, HID)])

    plsc.subcore_barrier()

    cps = [pltpu.async_copy(
        xpart_sh.at[pl.ds(i * HID + sid * LANES, LANES)],
        xs_v.at[pl.ds(i * LANES, LANES)], ex_sem)
        for i in range(NT)]
    for cp in cps:
        cp.wait()
    acc = xs_v[pl.ds(0, LANES)]
    for i in range(1, NT):
        acc = acc + xs_v[pl.ds(i * LANES, LANES)]
    pltpu.sync_copy(b1.at[pl.ds(sid * LANES, LANES)], b1s_v)
    pltpu.sync_copy(w2w.at[pl.ds(sid * LANES, LANES)], w2ws_v)
    v = jnp.clip(acc + b1s_v[...], 0.0, 127.0) * w2ws_v[...]
    pv_v[...] = v
    pltpu.sync_copy(pv_v, prod_sh.at[pl.ds(sid * LANES, LANES)])

    plsc.subcore_barrier()

    @pl.when(sid == 0)
    def _finale():
        pltpu.sync_copy(prod_sh, ps_v)
        acc = ps_v[pl.ds(0, LANES)]
        for i in range(1, NT):
            acc = acc + ps_v[pl.ds(i * LANES, LANES)]
        for s in (1, 2, 4, 8):
            acc = acc + _dyn_take(acc, lane_iota ^ s)
        pltpu.sync_copy(w2bp, w2b_v)
        out_v[...] = acc + w2b_v[...]
        pltpu.sync_copy(out_v, out_hbm)


@jax.jit
def _nnue_call(feats, w1f, b1, w2w, w2bp):
    mesh = plsc.VectorSubcoreMesh(core_axis_name="c", subcore_axis_name="s",
                                  num_cores=1)
    f = pl.kernel(
        _nnue_body,
        out_type=jax.ShapeDtypeStruct((LANES,), jnp.float32),
        mesh=mesh,
        compiler_params=pltpu.CompilerParams(needs_layout_passes=False),
        scratch_types=[
            pltpu.VMEM((IDX_PER_TILE,), jnp.int32),
            pltpu.VMEM((N_ROWS,), jnp.float32),
            pltpu.VMEM((NT * ROWS_PER_TILE,), jnp.float32),
            pltpu.VMEM((ROWS_PER_TILE,), jnp.float32),
            pltpu.VMEM((ROWS_PER_TILE * HID,), jnp.float32),
            pltpu.VMEM((HID,), jnp.float32),
            pltpu.VMEM((NT * LANES,), jnp.float32),
            pltpu.VMEM((LANES,), jnp.float32),
            pltpu.VMEM((LANES,), jnp.float32),
            pltpu.VMEM((LANES,), jnp.float32),
            pltpu.VMEM((NT * LANES,), jnp.float32),
            pltpu.VMEM((LANES,), jnp.float32),
            pltpu.VMEM((LANES,), jnp.float32),
            pltpu.VMEM_SHARED((NT * N_ROWS,), jnp.float32),
            pltpu.VMEM_SHARED((NT * HID,), jnp.float32),
            pltpu.VMEM_SHARED((NT * LANES,), jnp.float32),
            pltpu.SemaphoreType.DMA,
            pltpu.SemaphoreType.DMA,
        ],
    )
    return f(feats, w1f, b1, w2w, w2bp)


def kernel(feats, w1, b1, w2_w, w2_b):
    featsf = feats.astype(jnp.int32).reshape(-1)
    w1f = w1.reshape(-1)
    w2w = w2_w.reshape(HID)
    w2bp = jnp.pad(w2_b.astype(jnp.float32), (0, LANES - 1))
    res = _nnue_call(featsf, w1f, b1, w2w, w2bp)
    return res[0:1]


# speedup vs baseline: 1.0067x; 1.0067x over previous
import jax
import jax.numpy as jnp
from jax import lax
from jax.experimental import pallas as pl
from jax.experimental.pallas import tpu as pltpu
from jax.experimental.pallas import tpu_sc as plsc

FEATS_TOTAL = 16384
N_ROWS = 768
HID = 256
NT = 16
IDX_PER_TILE = FEATS_TOTAL // NT
ROWS_PER_TILE = N_ROWS // NT
LANES = 16
HB = 4
RHALF = ROWS_PER_TILE // 2


def _dyn_take(v, idx):
    dnums = lax.GatherDimensionNumbers(
        offset_dims=(), collapsed_slice_dims=(0,), start_index_map=(0,))
    return lax.gather(v, idx[:, None], dnums, slice_sizes=(1,),
                      mode=lax.GatherScatterMode.PROMISE_IN_BOUNDS)


def _nnue_body(feats3, w1f, b1, w2w, w2bp, out_hbm,
               idx_v, ones_v, zero_v, cw_v, w1_v, xpart_v,
               xall_v, b1_v, w2w_v, w2b_v, out_v,
               counts_sh, xpart_sh, w1_sem, st_sem):
    sid = lax.axis_index("s")
    lane_iota = lax.iota(jnp.int32, LANES)

    slab = sid * ROWS_PER_TILE * HID
    w1_cpa = pltpu.async_copy(
        w1f.at[pl.ds(slab, RHALF * HID)],
        w1_v.at[pl.ds(0, RHALF * HID)], w1_sem)
    w1_cpb = pltpu.async_copy(
        w1f.at[pl.ds(slab + RHALF * HID, RHALF * HID)],
        w1_v.at[pl.ds(RHALF * HID, RHALF * HID)], w1_sem)

    @pl.when(sid == 0)
    def _init():
        zeros16 = jnp.zeros((LANES,), jnp.float32)
        for i in range(N_ROWS // LANES):
            zero_v[pl.ds(i * LANES, LANES)] = zeros16
        pltpu.sync_copy(zero_v, counts_sh)

    ones16 = jnp.full((LANES,), 1.0, jnp.float32)
    for i in range(128 // LANES):
        ones_v[pl.ds(i * LANES, LANES)] = ones16

    plsc.subcore_barrier()

    pltpu.sync_copy(feats3.at[sid], idx_v)
    cps = [pltpu.async_copy(ones_v, counts_sh.at[idx_v.at[j]], st_sem,
                            add=True)
           for j in range(IDX_PER_TILE // 128)]
    for cp in cps:
        cp.wait()

    plsc.subcore_barrier()

    pltpu.sync_copy(counts_sh.at[pl.ds(sid * ROWS_PER_TILE,
                                       ROWS_PER_TILE)], cw_v)
    chunks = [cw_v[pl.ds(c * LANES, LANES)]
              for c in range(ROWS_PER_TILE // LANES)]
    w1_cpa.wait()
    w1_cpb.wait()
    for hb in range(HID // LANES // HB):
        accs = [jnp.zeros((LANES,), jnp.float32) for _ in range(HB)]
        for c in range(ROWS_PER_TILE // LANES):
            for r in range(LANES):
                bc = _dyn_take(chunks[c], jnp.full((LANES,), r, jnp.int32))
                base = (c * LANES + r) * HID
                for hh in range(HB):
                    accs[hh] = accs[hh] + bc * w1_v[
                        pl.ds(base + (hb * HB + hh) * LANES, LANES)]
        for hh in range(HB):
            xpart_v[pl.ds((hb * HB + hh) * LANES, LANES)] = accs[hh]
    pltpu.sync_copy(xpart_v, xpart_sh.at[pl.ds(sid * HID, HID)])

    plsc.subcore_barrier()

    @pl.when(sid == 0)
    def _finale():
        pltpu.sync_copy(xpart_sh, xall_v)
        pltpu.sync_copy(b1, b1_v)
        pltpu.sync_copy(w2w, w2w_v)
        pltpu.sync_copy(w2bp, w2b_v)
        acc = w2b_v[...]
        for h in range(HID // LANES):
            s = xall_v[pl.ds(h * LANES, LANES)]
            for i in range(1, NT):
                s = s + xall_v[pl.ds(i * HID + h * LANES, LANES)]
            v = jnp.clip(s + b1_v[pl.ds(h * LANES, LANES)], 0.0, 127.0)
            acc = acc + v * w2w_v[pl.ds(h * LANES, LANES)]
        for s2 in (1, 2, 4, 8):
            acc = acc + _dyn_take(acc, lane_iota ^ s2)
        out_v[...] = acc
        pltpu.sync_copy(out_v, out_hbm)


@jax.jit
def _nnue_call(feats3, w1f, b1, w2w, w2bp):
    mesh = plsc.VectorSubcoreMesh(core_axis_name="c", subcore_axis_name="s",
                                  num_cores=1)
    f = pl.kernel(
        _nnue_body,
        out_type=jax.ShapeDtypeStruct((LANES,), jnp.float32),
        mesh=mesh,
        scratch_types=[
            pltpu.VMEM((IDX_PER_TILE // 128, 128), jnp.int32),
            pltpu.VMEM((128,), jnp.float32),
            pltpu.VMEM((N_ROWS,), jnp.float32),
            pltpu.VMEM((ROWS_PER_TILE,), jnp.float32),
            pltpu.VMEM((ROWS_PER_TILE * HID,), jnp.float32),
            pltpu.VMEM((HID,), jnp.float32),
            pltpu.VMEM((NT * HID,), jnp.float32),
            pltpu.VMEM((HID,), jnp.float32),
            pltpu.VMEM((HID,), jnp.float32),
            pltpu.VMEM((LANES,), jnp.float32),
            pltpu.VMEM((LANES,), jnp.float32),
            pltpu.VMEM_SHARED((N_ROWS,), jnp.float32),
            pltpu.VMEM_SHARED((NT * HID,), jnp.float32),
            pltpu.SemaphoreType.DMA,
            pltpu.SemaphoreType.DMA,
        ],
    )
    return f(feats3, w1f, b1, w2w, w2bp)


def kernel(feats, w1, b1, w2_w, w2_b):
    feats3 = feats.astype(jnp.int32).reshape(NT, IDX_PER_TILE // 128, 128)
    w1f = w1.reshape(-1)
    w2w = w2_w.reshape(HID)
    w2bp = jnp.pad(w2_b.astype(jnp.float32), (0, LANES - 1))
    res = _nnue_call(feats3, w1f, b1, w2w, w2bp)
    return res[0:1]


# speedup vs baseline: 1.0990x; 1.0917x over previous
import jax
import jax.numpy as jnp
from jax import lax
from jax.experimental import pallas as pl
from jax.experimental.pallas import tpu as pltpu
from jax.experimental.pallas import tpu_sc as plsc

FEATS_TOTAL = 16384
N_ROWS = 768
HID = 256
NT = 16
IDX_PER_TILE = FEATS_TOTAL // NT
ROWS_PER_TILE = N_ROWS // NT
LANES = 16
HB = 4


def _dyn_take(v, idx):
    dnums = lax.GatherDimensionNumbers(
        offset_dims=(), collapsed_slice_dims=(0,), start_index_map=(0,))
    return lax.gather(v, idx[:, None], dnums, slice_sizes=(1,),
                      mode=lax.GatherScatterMode.PROMISE_IN_BOUNDS)


def _nnue_body(feats3, w1f, b1, w2w, w2bp, out_hbm,
               idx_v, ones_v, zero_v, cw_v, w1_v, xpart_v,
               idxa_v, idxb_v, b1_v, xq_v, w2w_v, w2b_v, out_v,
               counts_sh, x_sh, w1_sem, st_sem, z_sem, op_sem):
    sid = lax.axis_index("s")
    lane_iota = lax.iota(jnp.int32, LANES)

    w1_cp = pltpu.async_copy(
        w1f.at[pl.ds(sid * ROWS_PER_TILE * HID, ROWS_PER_TILE * HID)],
        w1_v, w1_sem)

    op_cps = []

    @pl.when(sid == 0)
    def _init():
        zeros16 = jnp.zeros((LANES,), jnp.float32)
        for i in range(N_ROWS // LANES):
            zero_v[pl.ds(i * LANES, LANES)] = zeros16
        zc = pltpu.async_copy(zero_v, counts_sh, z_sem)
        zx = pltpu.async_copy(zero_v.at[pl.ds(0, HID)], x_sh, z_sem)
        op_cps.append(pltpu.async_copy(b1, b1_v, op_sem))
        op_cps.append(pltpu.async_copy(w2w, w2w_v, op_sem))
        op_cps.append(pltpu.async_copy(w2bp, w2b_v, op_sem))
        zc.wait()
        zx.wait()

    for i in range(128 // LANES):
        idxa_v[pl.ds(i * LANES, LANES)] = lane_iota + (i * LANES)
        idxb_v[pl.ds(i * LANES, LANES)] = lane_iota + (128 + i * LANES)
    ones16 = jnp.full((LANES,), 1.0, jnp.float32)
    for i in range(128 // LANES):
        ones_v[pl.ds(i * LANES, LANES)] = ones16

    plsc.subcore_barrier()

    pltpu.sync_copy(feats3.at[sid], idx_v)
    cps = [pltpu.async_copy(ones_v, counts_sh.at[idx_v.at[j]], st_sem,
                            add=True)
           for j in range(IDX_PER_TILE // 128)]
    for cp in cps:
        cp.wait()

    plsc.subcore_barrier()

    pltpu.sync_copy(counts_sh.at[pl.ds(sid * ROWS_PER_TILE,
                                       ROWS_PER_TILE)], cw_v)
    w1_cp.wait()
    chunks = [cw_v[pl.ds(c * LANES, LANES)]
              for c in range(ROWS_PER_TILE // LANES)]
    for hb in range(HID // LANES // HB):
        accs = [jnp.zeros((LANES,), jnp.float32) for _ in range(HB)]
        for c in range(ROWS_PER_TILE // LANES):
            for r in range(LANES):
                bc = _dyn_take(chunks[c], jnp.full((LANES,), r, jnp.int32))
                base = (c * LANES + r) * HID
                for hh in range(HB):
                    accs[hh] = accs[hh] + bc * w1_v[
                        pl.ds(base + (hb * HB + hh) * LANES, LANES)]
        for hh in range(HB):
            xpart_v[pl.ds((hb * HB + hh) * LANES, LANES)] = accs[hh]
    pltpu.sync_copy(xpart_v.at[pl.ds(0, 128)], x_sh.at[idxa_v], add=True)
    pltpu.sync_copy(xpart_v.at[pl.ds(128, 128)], x_sh.at[idxb_v], add=True)

    plsc.subcore_barrier()

    @pl.when(sid == 0)
    def _finale():
        pltpu.sync_copy(x_sh, xq_v)
        for cp in op_cps:
            cp.wait()
        acc = w2b_v[...]
        for h in range(HID // LANES):
            v = jnp.clip(xq_v[pl.ds(h * LANES, LANES)]
                         + b1_v[pl.ds(h * LANES, LANES)], 0.0, 127.0)
            acc = acc + v * w2w_v[pl.ds(h * LANES, LANES)]
        for s in (1, 2, 4, 8):
            acc = acc + _dyn_take(acc, lane_iota ^ s)
        out_v[...] = acc
        pltpu.sync_copy(out_v, out_hbm)


@jax.jit
def _nnue_call(feats3, w1f, b1, w2w, w2bp):
    mesh = plsc.VectorSubcoreMesh(core_axis_name="c", subcore_axis_name="s",
                                  num_cores=1)
    f = pl.kernel(
        _nnue_body,
        out_type=jax.ShapeDtypeStruct((LANES,), jnp.float32),
        mesh=mesh,
        scratch_types=[
            pltpu.VMEM((IDX_PER_TILE // 128, 128), jnp.int32),
            pltpu.VMEM((128,), jnp.float32),
            pltpu.VMEM((N_ROWS,), jnp.float32),
            pltpu.VMEM((ROWS_PER_TILE,), jnp.float32),
            pltpu.VMEM((ROWS_PER_TILE * HID,), jnp.float32),
            pltpu.VMEM((HID,), jnp.float32),
            pltpu.VMEM((128,), jnp.int32),
            pltpu.VMEM((128,), jnp.int32),
            pltpu.VMEM((HID,), jnp.float32),
            pltpu.VMEM((HID,), jnp.float32),
            pltpu.VMEM((HID,), jnp.float32),
            pltpu.VMEM((LANES,), jnp.float32),
            pltpu.VMEM((LANES,), jnp.float32),
            pltpu.VMEM_SHARED((N_ROWS,), jnp.float32),
            pltpu.VMEM_SHARED((HID,), jnp.float32),
            pltpu.SemaphoreType.DMA,
            pltpu.SemaphoreType.DMA,
            pltpu.SemaphoreType.DMA,
            pltpu.SemaphoreType.DMA,
        ],
    )
    return f(feats3, w1f, b1, w2w, w2bp)


def kernel(feats, w1, b1, w2_w, w2_b):
    feats3 = feats.astype(jnp.int32).reshape(NT, IDX_PER_TILE // 128, 128)
    w1f = w1.reshape(-1)
    w2w = w2_w.reshape(HID)
    w2bp = jnp.pad(w2_b.astype(jnp.float32), (0, LANES - 1))
    res = _nnue_call(feats3, w1f, b1, w2w, w2bp)
    return res[0:1]


# speedup vs baseline: 1.1319x; 1.0299x over previous
import jax
import jax.numpy as jnp
from jax import lax
from jax.experimental import pallas as pl
from jax.experimental.pallas import tpu as pltpu
from jax.experimental.pallas import tpu_sc as plsc

FEATS_TOTAL = 16384
N_ROWS = 768
HID = 256
NT = 16
IDX_PER_TILE = FEATS_TOTAL // NT
ROWS_PER_TILE = N_ROWS // NT
LANES = 16
HB = 4


def _dyn_take(v, idx):
    dnums = lax.GatherDimensionNumbers(
        offset_dims=(), collapsed_slice_dims=(0,), start_index_map=(0,))
    return lax.gather(v, idx[:, None], dnums, slice_sizes=(1,),
                      mode=lax.GatherScatterMode.PROMISE_IN_BOUNDS)


def _nnue_body(feats3, w1f, b1, w2w, w2bp, out_hbm,
               idx_v, ones_v, zero_v, cw_v, w1_v, xpart_v,
               idxa_v, idxb_v, b1_v, xq_v, w2w_v, w2b_v, out_v,
               counts_sh, x_sh, w1_sem, st_sem, z_sem, op_sem):
    sid = lax.axis_index("s")
    lane_iota = lax.iota(jnp.int32, LANES)

    slab = sid * ROWS_PER_TILE * HID
    csz = (ROWS_PER_TILE // 3) * HID
    w1_cps = [pltpu.async_copy(
        w1f.at[pl.ds(slab + c * csz, csz)],
        w1_v.at[pl.ds(c * csz, csz)], w1_sem)
        for c in range(3)]

    op_cps = []

    @pl.when(sid == 0)
    def _init():
        zeros16 = jnp.zeros((LANES,), jnp.float32)
        for i in range(N_ROWS // LANES):
            zero_v[pl.ds(i * LANES, LANES)] = zeros16
        zc = pltpu.async_copy(zero_v, counts_sh, z_sem)
        zx = pltpu.async_copy(zero_v.at[pl.ds(0, HID)], x_sh, z_sem)
        op_cps.append(pltpu.async_copy(b1, b1_v, op_sem))
        op_cps.append(pltpu.async_copy(w2w, w2w_v, op_sem))
        op_cps.append(pltpu.async_copy(w2bp, w2b_v, op_sem))
        zc.wait()
        zx.wait()

    for i in range(128 // LANES):
        idxa_v[pl.ds(i * LANES, LANES)] = lane_iota + (i * LANES)
        idxb_v[pl.ds(i * LANES, LANES)] = lane_iota + (128 + i * LANES)
    ones16 = jnp.full((LANES,), 1.0, jnp.float32)
    for i in range(128 // LANES):
        ones_v[pl.ds(i * LANES, LANES)] = ones16

    plsc.subcore_barrier()

    pltpu.sync_copy(feats3.at[sid], idx_v)
    cps = [pltpu.async_copy(ones_v, counts_sh.at[idx_v.at[j]], st_sem,
                            add=True)
           for j in range(IDX_PER_TILE // 128)]
    for cp in cps:
        cp.wait()

    plsc.subcore_barrier()

    pltpu.sync_copy(counts_sh.at[pl.ds(sid * ROWS_PER_TILE,
                                       ROWS_PER_TILE)], cw_v)
    chunks = [cw_v[pl.ds(c * LANES, LANES)]
              for c in range(ROWS_PER_TILE // LANES)]
    x_cps = []
    for hb in range(HID // LANES // HB):
        accs = [jnp.zeros((LANES,), jnp.float32) for _ in range(HB)]
        for c in range(ROWS_PER_TILE // LANES):
            if hb == 0:
                w1_cps[c].wait()
            for r in range(LANES):
                row = c * LANES + r
                s = chunks[c][r]
                base = row * HID
                for hh in range(HB):
                    accs[hh] = accs[hh] + s * w1_v[
                        pl.ds(base + (hb * HB + hh) * LANES, LANES)]
        for hh in range(HB):
            xpart_v[pl.ds((hb * HB + hh) * LANES, LANES)] = accs[hh]
        if hb == 1:
            x_cps.append(pltpu.async_copy(
                xpart_v.at[pl.ds(0, 128)], x_sh.at[idxa_v], st_sem,
                add=True))
        if hb == 3:
            x_cps.append(pltpu.async_copy(
                xpart_v.at[pl.ds(128, 128)], x_sh.at[idxb_v], st_sem,
                add=True))
    for cp in x_cps:
        cp.wait()

    plsc.subcore_barrier()

    @pl.when(sid == 0)
    def _finale():
        pltpu.sync_copy(x_sh, xq_v)
        for cp in op_cps:
            cp.wait()
        acc = w2b_v[...]
        for h in range(HID // LANES):
            v = jnp.clip(xq_v[pl.ds(h * LANES, LANES)]
                         + b1_v[pl.ds(h * LANES, LANES)], 0.0, 127.0)
            acc = acc + v * w2w_v[pl.ds(h * LANES, LANES)]
        for s in (1, 2, 4, 8):
            acc = acc + _dyn_take(acc, lane_iota ^ s)
        out_v[...] = acc
        pltpu.sync_copy(out_v, out_hbm)


@jax.jit
def _nnue_call(feats3, w1f, b1, w2w, w2bp):
    mesh = plsc.VectorSubcoreMesh(core_axis_name="c", subcore_axis_name="s",
                                  num_cores=1)
    f = pl.kernel(
        _nnue_body,
        out_type=jax.ShapeDtypeStruct((LANES,), jnp.float32),
        mesh=mesh,
        scratch_types=[
            pltpu.VMEM((IDX_PER_TILE // 128, 128), jnp.int32),
            pltpu.VMEM((128,), jnp.float32),
            pltpu.VMEM((N_ROWS,), jnp.float32),
            pltpu.VMEM((ROWS_PER_TILE,), jnp.float32),
            pltpu.VMEM((ROWS_PER_TILE * HID,), jnp.float32),
            pltpu.VMEM((HID,), jnp.float32),
            pltpu.VMEM((128,), jnp.int32),
            pltpu.VMEM((128,), jnp.int32),
            pltpu.VMEM((HID,), jnp.float32),
            pltpu.VMEM((HID,), jnp.float32),
            pltpu.VMEM((HID,), jnp.float32),
            pltpu.VMEM((LANES,), jnp.float32),
            pltpu.VMEM((LANES,), jnp.float32),
            pltpu.VMEM_SHARED((N_ROWS,), jnp.float32),
            pltpu.VMEM_SHARED((HID,), jnp.float32),
            pltpu.SemaphoreType.DMA,
            pltpu.SemaphoreType.DMA,
            pltpu.SemaphoreType.DMA,
            pltpu.SemaphoreType.DMA,
        ],
    )
    return f(feats3, w1f, b1, w2w, w2bp)


def kernel(feats, w1, b1, w2_w, w2_b):
    feats3 = feats.astype(jnp.int32).reshape(NT, IDX_PER_TILE // 128, 128)
    w1f = w1.reshape(-1)
    w2w = w2_w.reshape(HID)
    w2bp = jnp.pad(w2_b.astype(jnp.float32), (0, LANES - 1))
    res = _nnue_call(feats3, w1f, b1, w2w, w2bp)
    return res[0:1]


# speedup vs baseline: 1.1348x; 1.0026x over previous
import jax
import jax.numpy as jnp
from jax import lax
from jax.experimental import pallas as pl
from jax.experimental.pallas import tpu as pltpu
from jax.experimental.pallas import tpu_sc as plsc

FEATS_TOTAL = 16384
N_ROWS = 768
HID = 256
NT = 16
IDX_PER_TILE = FEATS_TOTAL // NT
ROWS_PER_TILE = N_ROWS // NT
LANES = 16
HB = 8


def _dyn_take(v, idx):
    dnums = lax.GatherDimensionNumbers(
        offset_dims=(), collapsed_slice_dims=(0,), start_index_map=(0,))
    return lax.gather(v, idx[:, None], dnums, slice_sizes=(1,),
                      mode=lax.GatherScatterMode.PROMISE_IN_BOUNDS)


def _nnue_body(feats3, w1f, b1, w2w, w2bp, out_hbm,
               idx_v, ones_v, zero_v, cw_v, w1_v, xpart_v,
               idxa_v, idxb_v, b1_v, xq_v, w2w_v, w2b_v, out_v,
               counts_sh, x_sh, w1_sem, st_sem, z_sem, op_sem):
    sid = lax.axis_index("s")
    lane_iota = lax.iota(jnp.int32, LANES)

    slab = sid * ROWS_PER_TILE * HID
    csz = (ROWS_PER_TILE // 3) * HID
    w1_cps = [pltpu.async_copy(
        w1f.at[pl.ds(slab + c * csz, csz)],
        w1_v.at[pl.ds(c * csz, csz)], w1_sem)
        for c in range(3)]

    op_cps = []

    @pl.when(sid == 0)
    def _init():
        zeros16 = jnp.zeros((LANES,), jnp.float32)
        for i in range(N_ROWS // LANES):
            zero_v[pl.ds(i * LANES, LANES)] = zeros16
        zc = pltpu.async_copy(zero_v, counts_sh, z_sem)
        zx = pltpu.async_copy(zero_v.at[pl.ds(0, HID)], x_sh, z_sem)
        op_cps.append(pltpu.async_copy(b1, b1_v, op_sem))
        op_cps.append(pltpu.async_copy(w2w, w2w_v, op_sem))
        op_cps.append(pltpu.async_copy(w2bp, w2b_v, op_sem))
        zc.wait()
        zx.wait()

    for i in range(128 // LANES):
        idxa_v[pl.ds(i * LANES, LANES)] = lane_iota + (i * LANES)
        idxb_v[pl.ds(i * LANES, LANES)] = lane_iota + (128 + i * LANES)
    ones16 = jnp.full((LANES,), 1.0, jnp.float32)
    for i in range(128 // LANES):
        ones_v[pl.ds(i * LANES, LANES)] = ones16

    plsc.subcore_barrier()

    pltpu.sync_copy(feats3.at[sid], idx_v)
    cps = [pltpu.async_copy(ones_v, counts_sh.at[idx_v.at[j]], st_sem,
                            add=True)
           for j in range(IDX_PER_TILE // 128)]
    for cp in cps:
        cp.wait()

    plsc.subcore_barrier()

    pltpu.sync_copy(counts_sh.at[pl.ds(sid * ROWS_PER_TILE,
                                       ROWS_PER_TILE)], cw_v)
    chunks = [cw_v[pl.ds(c * LANES, LANES)]
              for c in range(ROWS_PER_TILE // LANES)]
    x_cps = []
    for hb in range(HID // LANES // HB):
        accs = [jnp.zeros((LANES,), jnp.float32) for _ in range(HB)]
        for c in range(ROWS_PER_TILE // LANES):
            if hb == 0:
                w1_cps[c].wait()
            for r in range(LANES):
                row = c * LANES + r
                s = chunks[c][r]
                base = row * HID
                for hh in range(HB):
                    accs[hh] = accs[hh] + s * w1_v[
                        pl.ds(base + (hb * HB + hh) * LANES, LANES)]
        for hh in range(HB):
            xpart_v[pl.ds((hb * HB + hh) * LANES, LANES)] = accs[hh]
        if hb == 0:
            x_cps.append(pltpu.async_copy(
                xpart_v.at[pl.ds(0, 128)], x_sh.at[idxa_v], st_sem,
                add=True))
        if hb == 1:
            x_cps.append(pltpu.async_copy(
                xpart_v.at[pl.ds(128, 128)], x_sh.at[idxb_v], st_sem,
                add=True))
    for cp in x_cps:
        cp.wait()

    plsc.subcore_barrier()

    @pl.when(sid == 0)
    def _finale():
        pltpu.sync_copy(x_sh, xq_v)
        for cp in op_cps:
            cp.wait()
        acc = w2b_v[...]
        for h in range(HID // LANES):
            v = jnp.clip(xq_v[pl.ds(h * LANES, LANES)]
                         + b1_v[pl.ds(h * LANES, LANES)], 0.0, 127.0)
            acc = acc + v * w2w_v[pl.ds(h * LANES, LANES)]
        for s in (1, 2, 4, 8):
            acc = acc + _dyn_take(acc, lane_iota ^ s)
        out_v[...] = acc
        pltpu.sync_copy(out_v, out_hbm)


@jax.jit
def _nnue_call(feats3, w1f, b1, w2w, w2bp):
    mesh = plsc.VectorSubcoreMesh(core_axis_name="c", subcore_axis_name="s",
                                  num_cores=1)
    f = pl.kernel(
        _nnue_body,
        out_type=jax.ShapeDtypeStruct((LANES,), jnp.float32),
        mesh=mesh,
        scratch_types=[
            pltpu.VMEM((IDX_PER_TILE // 128, 128), jnp.int32),
            pltpu.VMEM((128,), jnp.float32),
            pltpu.VMEM((N_ROWS,), jnp.float32),
            pltpu.VMEM((ROWS_PER_TILE,), jnp.float32),
            pltpu.VMEM((ROWS_PER_TILE * HID,), jnp.float32),
            pltpu.VMEM((HID,), jnp.float32),
            pltpu.VMEM((128,), jnp.int32),
            pltpu.VMEM((128,), jnp.int32),
            pltpu.VMEM((HID,), jnp.float32),
            pltpu.VMEM((HID,), jnp.float32),
            pltpu.VMEM((HID,), jnp.float32),
            pltpu.VMEM((LANES,), jnp.float32),
            pltpu.VMEM((LANES,), jnp.float32),
            pltpu.VMEM_SHARED((N_ROWS,), jnp.float32),
            pltpu.VMEM_SHARED((HID,), jnp.float32),
            pltpu.SemaphoreType.DMA,
            pltpu.SemaphoreType.DMA,
            pltpu.SemaphoreType.DMA,
            pltpu.SemaphoreType.DMA,
        ],
    )
    return f(feats3, w1f, b1, w2w, w2bp)


def kernel(feats, w1, b1, w2_w, w2_b):
    feats3 = feats.astype(jnp.int32).reshape(NT, IDX_PER_TILE // 128, 128)
    w1f = w1.reshape(-1)
    w2w = w2_w.reshape(HID)
    w2bp = jnp.pad(w2_b.astype(jnp.float32), (0, LANES - 1))
    res = _nnue_call(feats3, w1f, b1, w2w, w2bp)
    return res[0:1]


# speedup vs baseline: 1.1476x; 1.0113x over previous
import jax
import jax.numpy as jnp
from jax import lax
from jax.experimental import pallas as pl
from jax.experimental.pallas import tpu as pltpu
from jax.experimental.pallas import tpu_sc as plsc

FEATS_TOTAL = 16384
N_ROWS = 768
HID = 256
NT = 16
IDX_PER_TILE = FEATS_TOTAL // NT
ROWS_PER_TILE = N_ROWS // NT
LANES = 16
HB = 8


def _dyn_take(v, idx):
    dnums = lax.GatherDimensionNumbers(
        offset_dims=(), collapsed_slice_dims=(0,), start_index_map=(0,))
    return lax.gather(v, idx[:, None], dnums, slice_sizes=(1,),
                      mode=lax.GatherScatterMode.PROMISE_IN_BOUNDS)


def _nnue_body(feats3, w1f, b1, w2w, w2bp, out_hbm,
               idx_v, ones_v, zero_v, cw_v, w1_v, xpart_v,
               idxa_v, idxb_v, b1_v, xq_v, w2w_v, w2b_v, out_v,
               counts_sh, x_sh, w1_sem, st_sem, z_sem, op_sem, f_sem):
    sid = lax.axis_index("s")
    lane_iota = lax.iota(jnp.int32, LANES)

    slab = sid * ROWS_PER_TILE * HID
    csz = (ROWS_PER_TILE // 3) * HID
    w1_cps = [pltpu.async_copy(
        w1f.at[pl.ds(slab + c * csz, csz)],
        w1_v.at[pl.ds(c * csz, csz)], w1_sem)
        for c in range(3)]
    f_cp = pltpu.async_copy(feats3.at[sid], idx_v, f_sem)

    op_cps = []

    @pl.when(sid == 0)
    def _init():
        zeros16 = jnp.zeros((LANES,), jnp.float32)
        for i in range(N_ROWS // LANES):
            zero_v[pl.ds(i * LANES, LANES)] = zeros16
        zc = pltpu.async_copy(zero_v, counts_sh, z_sem)
        zx = pltpu.async_copy(zero_v.at[pl.ds(0, HID)], x_sh, z_sem)
        op_cps.append(pltpu.async_copy(b1, b1_v, op_sem))
        op_cps.append(pltpu.async_copy(w2w, w2w_v, op_sem))
        op_cps.append(pltpu.async_copy(w2bp, w2b_v, op_sem))
        zc.wait()
        zx.wait()

    for i in range(128 // LANES):
        idxa_v[pl.ds(i * LANES, LANES)] = lane_iota + (i * LANES)
        idxb_v[pl.ds(i * LANES, LANES)] = lane_iota + (128 + i * LANES)
    ones16 = jnp.full((LANES,), 1.0, jnp.float32)
    for i in range(128 // LANES):
        ones_v[pl.ds(i * LANES, LANES)] = ones16

    plsc.subcore_barrier()

    f_cp.wait()
    cps = [pltpu.async_copy(ones_v, counts_sh.at[idx_v.at[j]], st_sem,
                            add=True)
           for j in range(IDX_PER_TILE // 128)]
    for cp in cps:
        cp.wait()

    plsc.subcore_barrier()

    pltpu.sync_copy(counts_sh.at[pl.ds(sid * ROWS_PER_TILE,
                                       ROWS_PER_TILE)], cw_v)
    chunks = [cw_v[pl.ds(c * LANES, LANES)]
              for c in range(ROWS_PER_TILE // LANES)]
    x_cps = []
    for hb in range(HID // LANES // HB):
        accs = [jnp.zeros((LANES,), jnp.float32) for _ in range(HB)]
        for c in range(ROWS_PER_TILE // LANES):
            if hb == 0:
                w1_cps[c].wait()
            for r in range(LANES):
                row = c * LANES + r
                s = chunks[c][r]
                base = row * HID
                for hh in range(HB):
                    accs[hh] = accs[hh] + s * w1_v[
                        pl.ds(base + (hb * HB + hh) * LANES, LANES)]
        for hh in range(HB):
            xpart_v[pl.ds((hb * HB + hh) * LANES, LANES)] = accs[hh]
        if hb == 0:
            x_cps.append(pltpu.async_copy(
                xpart_v.at[pl.ds(0, 128)], x_sh.at[idxa_v], st_sem,
                add=True))
        if hb == 1:
            x_cps.append(pltpu.async_copy(
                xpart_v.at[pl.ds(128, 128)], x_sh.at[idxb_v], st_sem,
                add=True))
    for cp in x_cps:
        cp.wait()

    plsc.subcore_barrier()

    @pl.when(sid == 0)
    def _finale():
        pltpu.sync_copy(x_sh, xq_v)
        for cp in op_cps:
            cp.wait()
        acc = w2b_v[...]
        for h in range(HID // LANES):
            v = jnp.clip(xq_v[pl.ds(h * LANES, LANES)]
                         + b1_v[pl.ds(h * LANES, LANES)], 0.0, 127.0)
            acc = acc + v * w2w_v[pl.ds(h * LANES, LANES)]
        for s in (1, 2, 4, 8):
            acc = acc + _dyn_take(acc, lane_iota ^ s)
        out_v[...] = acc
        pltpu.sync_copy(out_v, out_hbm)


@jax.jit
def _nnue_call(feats3, w1f, b1, w2w, w2bp):
    mesh = plsc.VectorSubcoreMesh(core_axis_name="c", subcore_axis_name="s",
                                  num_cores=1)
    f = pl.kernel(
        _nnue_body,
        out_type=jax.ShapeDtypeStruct((LANES,), jnp.float32),
        mesh=mesh,
        scratch_types=[
            pltpu.VMEM((IDX_PER_TILE // 128, 128), jnp.int32),
            pltpu.VMEM((128,), jnp.float32),
            pltpu.VMEM((N_ROWS,), jnp.float32),
            pltpu.VMEM((ROWS_PER_TILE,), jnp.float32),
            pltpu.VMEM((ROWS_PER_TILE * HID,), jnp.float32),
            pltpu.VMEM((HID,), jnp.float32),
            pltpu.VMEM((128,), jnp.int32),
            pltpu.VMEM((128,), jnp.int32),
            pltpu.VMEM((HID,), jnp.float32),
            pltpu.VMEM((HID,), jnp.float32),
            pltpu.VMEM((HID,), jnp.float32),
            pltpu.VMEM((LANES,), jnp.float32),
            pltpu.VMEM((LANES,), jnp.float32),
            pltpu.VMEM_SHARED((N_ROWS,), jnp.float32),
            pltpu.VMEM_SHARED((HID,), jnp.float32),
            pltpu.SemaphoreType.DMA,
            pltpu.SemaphoreType.DMA,
            pltpu.SemaphoreType.DMA,
            pltpu.SemaphoreType.DMA,
            pltpu.SemaphoreType.DMA,
        ],
    )
    return f(feats3, w1f, b1, w2w, w2bp)


def kernel(feats, w1, b1, w2_w, w2_b):
    feats3 = feats.astype(jnp.int32).reshape(NT, IDX_PER_TILE // 128, 128)
    w1f = w1.reshape(-1)
    w2w = w2_w.reshape(HID)
    w2bp = jnp.pad(w2_b.astype(jnp.float32), (0, LANES - 1))
    res = _nnue_call(feats3, w1f, b1, w2w, w2bp)
    return res[0:1]


# speedup vs baseline: 1.1558x; 1.0072x over previous
import jax
import jax.numpy as jnp
from jax import lax
from jax.experimental import pallas as pl
from jax.experimental.pallas import tpu as pltpu
from jax.experimental.pallas import tpu_sc as plsc

FEATS_TOTAL = 16384
N_ROWS = 768
HID = 256
NT = 16
IDX_PER_TILE = FEATS_TOTAL // NT
ROWS_PER_TILE = N_ROWS // NT
LANES = 16
HB = 8


def _dyn_take(v, idx):
    dnums = lax.GatherDimensionNumbers(
        offset_dims=(), collapsed_slice_dims=(0,), start_index_map=(0,))
    return lax.gather(v, idx[:, None], dnums, slice_sizes=(1,),
                      mode=lax.GatherScatterMode.PROMISE_IN_BOUNDS)


def _nnue_body(feats3, w1f, b1, w2w, w2bp, out_hbm,
               idx_v, ones_v, zero_v, cw_v, w1_v, xpart_v,
               idxa_v, idxb_v, b1_v, xq_v, w2w_v, w2b_v, out_v,
               counts_sh, x_sh, w1_sem, st_sem, z_sem, op_sem, f_sem):
    sid = lax.axis_index("s")
    lane_iota = lax.iota(jnp.int32, LANES)

    slab = sid * ROWS_PER_TILE * HID
    NW = 1
    csz = (ROWS_PER_TILE // NW) * HID
    w1_cps = [pltpu.async_copy(
        w1f.at[pl.ds(slab + c * csz, csz)],
        w1_v.at[pl.ds(c * csz, csz)], w1_sem)
        for c in range(NW)]
    f_cp = pltpu.async_copy(feats3.at[sid], idx_v, f_sem)

    op_cps = []

    @pl.when(sid == 0)
    def _init():
        zeros16 = jnp.zeros((LANES,), jnp.float32)
        for i in range(N_ROWS // LANES):
            zero_v[pl.ds(i * LANES, LANES)] = zeros16
        zc = pltpu.async_copy(zero_v, counts_sh, z_sem)
        zx = pltpu.async_copy(zero_v.at[pl.ds(0, HID)], x_sh, z_sem)
        op_cps.append(pltpu.async_copy(b1, b1_v, op_sem))
        op_cps.append(pltpu.async_copy(w2w, w2w_v, op_sem))
        op_cps.append(pltpu.async_copy(w2bp, w2b_v, op_sem))
        zc.wait()
        zx.wait()

    for i in range(128 // LANES):
        idxa_v[pl.ds(i * LANES, LANES)] = lane_iota + (i * LANES)
        idxb_v[pl.ds(i * LANES, LANES)] = lane_iota + (128 + i * LANES)
    ones16 = jnp.full((LANES,), 1.0, jnp.float32)
    for i in range(128 // LANES):
        ones_v[pl.ds(i * LANES, LANES)] = ones16

    plsc.subcore_barrier()

    f_cp.wait()
    cps = [pltpu.async_copy(ones_v, counts_sh.at[idx_v.at[j]], st_sem,
                            add=True)
           for j in range(IDX_PER_TILE // 128)]
    for cp in cps:
        cp.wait()

    plsc.subcore_barrier()

    pltpu.sync_copy(counts_sh.at[pl.ds(sid * ROWS_PER_TILE,
                                       ROWS_PER_TILE)], cw_v)
    chunks = [cw_v[pl.ds(c * LANES, LANES)]
              for c in range(ROWS_PER_TILE // LANES)]
    x_cps = []
    for hb in range(HID // LANES // HB):
        accs = [jnp.zeros((LANES,), jnp.float32) for _ in range(HB)]
        for c in range(ROWS_PER_TILE // LANES):
            if hb == 0 and c < NW:
                w1_cps[c].wait()
            for r in range(LANES):
                row = c * LANES + r
                s = chunks[c][r]
                base = row * HID
                for hh in range(HB):
                    accs[hh] = accs[hh] + s * w1_v[
                        pl.ds(base + (hb * HB + hh) * LANES, LANES)]
        for hh in range(HB):
            xpart_v[pl.ds((hb * HB + hh) * LANES, LANES)] = accs[hh]
        if hb == 0:
            x_cps.append(pltpu.async_copy(
                xpart_v.at[pl.ds(0, 128)], x_sh.at[idxa_v], st_sem,
                add=True))
        if hb == 1:
            x_cps.append(pltpu.async_copy(
                xpart_v.at[pl.ds(128, 128)], x_sh.at[idxb_v], st_sem,
                add=True))
    for cp in x_cps:
        cp.wait()

    plsc.subcore_barrier()

    @pl.when(sid == 0)
    def _finale():
        pltpu.sync_copy(x_sh, xq_v)
        for cp in op_cps:
            cp.wait()
        acc = w2b_v[...]
        for h in range(HID // LANES):
            v = jnp.clip(xq_v[pl.ds(h * LANES, LANES)]
                         + b1_v[pl.ds(h * LANES, LANES)], 0.0, 127.0)
            acc = acc + v * w2w_v[pl.ds(h * LANES, LANES)]
        for s in (1, 2, 4, 8):
            acc = acc + _dyn_take(acc, lane_iota ^ s)
        out_v[...] = acc
        pltpu.sync_copy(out_v, out_hbm)


@jax.jit
def _nnue_call(feats3, w1f, b1, w2w, w2bp):
    mesh = plsc.VectorSubcoreMesh(core_axis_name="c", subcore_axis_name="s",
                                  num_cores=1)
    f = pl.kernel(
        _nnue_body,
        out_type=jax.ShapeDtypeStruct((LANES,), jnp.float32),
        mesh=mesh,
        scratch_types=[
            pltpu.VMEM((IDX_PER_TILE // 128, 128), jnp.int32),
            pltpu.VMEM((128,), jnp.float32),
            pltpu.VMEM((N_ROWS,), jnp.float32),
            pltpu.VMEM((ROWS_PER_TILE,), jnp.float32),
            pltpu.VMEM((ROWS_PER_TILE * HID,), jnp.float32),
            pltpu.VMEM((HID,), jnp.float32),
            pltpu.VMEM((128,), jnp.int32),
            pltpu.VMEM((128,), jnp.int32),
            pltpu.VMEM((HID,), jnp.float32),
            pltpu.VMEM((HID,), jnp.float32),
            pltpu.VMEM((HID,), jnp.float32),
            pltpu.VMEM((LANES,), jnp.float32),
            pltpu.VMEM((LANES,), jnp.float32),
            pltpu.VMEM_SHARED((N_ROWS,), jnp.float32),
            pltpu.VMEM_SHARED((HID,), jnp.float32),
            pltpu.SemaphoreType.DMA,
            pltpu.SemaphoreType.DMA,
            pltpu.SemaphoreType.DMA,
            pltpu.SemaphoreType.DMA,
            pltpu.SemaphoreType.DMA,
        ],
    )
    return f(feats3, w1f, b1, w2w, w2bp)


def kernel(feats, w1, b1, w2_w, w2_b):
    feats3 = feats.astype(jnp.int32).reshape(NT, IDX_PER_TILE // 128, 128)
    w1f = w1.reshape(-1)
    w2w = w2_w.reshape(HID)
    w2bp = jnp.pad(w2_b.astype(jnp.float32), (0, LANES - 1))
    res = _nnue_call(feats3, w1f, b1, w2w, w2bp)
    return res[0:1]
